# baseline (speedup 1.0000x reference)
import functools

import jax
import jax.numpy as jnp
from jax import lax
from jax.experimental import pallas as pl
from jax.experimental.pallas import tpu as pltpu
from jax.experimental.pallas import tpu_sc as plsc

N = 10000
E = 320000
D = 128

NC = 2
NS = 16
L = 16

_mesh = plsc.VectorSubcoreMesh(core_axis_name="c", subcore_axis_name="s")
_sc_params = pltpu.CompilerParams(needs_layout_passes=False)

HSZ = 16384
EPT_DEG = E // NS
HPT = HSZ // NS


@functools.partial(
    pl.kernel,
    out_type=[
        jax.ShapeDtypeStruct((HSZ,), jnp.float32),
        jax.ShapeDtypeStruct((HSZ,), jnp.float32),
    ],
    mesh=_mesh,
    scratch_types=[
        pltpu.VMEM((EPT_DEG,), jnp.int32),
        pltpu.VMEM((HSZ,), jnp.float32),
        pltpu.VMEM((HPT,), jnp.float32),
        pltpu.VMEM((HPT,), jnp.float32),
        pltpu.VMEM_SHARED((NS * HSZ,), jnp.float32),
    ],
    compiler_params=_sc_params,
)
def _deg_kernel(src_hbm, dst_hbm, dsrc_hbm, ddst_hbm,
                idx_v, hist_v, acc_v, tmp_v, shared):
    c = lax.axis_index("c")
    s = lax.axis_index("s")

    zeros16 = jnp.zeros((L,), jnp.float32)
    ones16 = jnp.ones((L,), jnp.float32)

    def zero_hist(i, carry):
        hist_v[pl.ds(i * L, L)] = zeros16
        return carry

    lax.fori_loop(0, HSZ // L, zero_hist, 0)

    @pl.when(c == 0)
    def _():
        pltpu.sync_copy(src_hbm.at[pl.ds(s * EPT_DEG, EPT_DEG)], idx_v)

    @pl.when(c == 1)
    def _():
        pltpu.sync_copy(dst_hbm.at[pl.ds(s * EPT_DEG, EPT_DEG)], idx_v)

    def accum(i, carry):
        idx = idx_v[pl.ds(i * L, L)]
        plsc.addupdate_scatter(hist_v, [idx], ones16)
        return carry

    lax.fori_loop(0, EPT_DEG // L, accum, 0)

    pltpu.sync_copy(hist_v, shared.at[pl.ds(s * HSZ, HSZ)])
    plsc.subcore_barrier()

    def zero_acc(i, carry):
        acc_v[pl.ds(i * L, L)] = zeros16
        return carry

    lax.fori_loop(0, HPT // L, zero_acc, 0)

    def combine(k, carry):
        pltpu.sync_copy(shared.at[pl.ds(k * HSZ + s * HPT, HPT)], tmp_v)

        def add_vec(i, carry2):
            j = i * L
            acc_v[pl.ds(j, L)] = acc_v[pl.ds(j, L)] + tmp_v[pl.ds(j, L)]
            return carry2

        lax.fori_loop(0, HPT // L, add_vec, 0)
        return carry

    lax.fori_loop(0, NS, combine, 0)

    @pl.when(c == 0)
    def _():
        pltpu.sync_copy(acc_v, dsrc_hbm.at[pl.ds(s * HPT, HPT)])

    @pl.when(c == 1)
    def _():
        pltpu.sync_copy(acc_v, ddst_hbm.at[pl.ds(s * HPT, HPT)])


RMM = 1000


def _mm_body(f_ref, w_ref, deg_ref, xw_ref):
    scale = lax.rsqrt(jnp.maximum(deg_ref[...], 1.0))
    xw_ref[...] = jnp.dot(f_ref[...], w_ref[...],
                          preferred_element_type=jnp.float32) * scale


def _mm(features, W, deg_out2d):
    return pl.pallas_call(
        _mm_body,
        grid=(N // RMM,),
        in_specs=[
            pl.BlockSpec((RMM, D), lambda i: (i, 0)),
            pl.BlockSpec((D, D), lambda i: (0, 0)),
            pl.BlockSpec((RMM, 1), lambda i: (i, 0)),
        ],
        out_specs=pl.BlockSpec((RMM, D), lambda i: (i, 0)),
        out_shape=jax.ShapeDtypeStruct((N, D), jnp.float32),
    )(features, W, deg_out2d)


BB = 80
EPT = E // (NC * NS)
CAP = EPT + 240
HALF = 5000
AGG = 6144
TRASH = 5632
ZR = 48
OPT = 5120 // NS
PBITS = 14
PMASK = (1 << PBITS) - 1


@functools.partial(
    pl.kernel,
    out_type=jax.ShapeDtypeStruct((2, NC, 5120, D), jnp.float32),
    mesh=_mesh,
    scratch_types=[
        pltpu.VMEM((CAP,), jnp.int32),
        pltpu.VMEM((CAP,), jnp.int32),
        pltpu.VMEM((4, BB), jnp.int32),
        pltpu.VMEM((4, BB), jnp.int32),
        pltpu.VMEM((BB, D), jnp.float32),
        pltpu.VMEM((BB, D), jnp.float32),
        pltpu.VMEM((BB, D), jnp.float32),
        pltpu.VMEM((BB, D), jnp.float32),
        pltpu.VMEM_SHARED((AGG, D), jnp.float32),
        pltpu.SemaphoreType.DMA,
        pltpu.SemaphoreType.DMA,
        pltpu.SemaphoreType.DMA,
        pltpu.SemaphoreType.DMA,
        pltpu.SemaphoreType.DMA,
        pltpu.SemaphoreType.DMA,
        pltpu.SemaphoreType.DMA,
        pltpu.SemaphoreType.DMA,
    ],
    compiler_params=_sc_params,
)
def _gs_kernel(xw_hbm, src_hbm, dst_hbm, out_hbm,
               lo_v, hi_v, srcB, dstB, r0_v, r1_v, r2_v, r3_v,
               shared, sg0, sg1, sg2, sg3, ss0, ss1, ss2, ss3):
    c = lax.axis_index("c")
    s = lax.axis_index("s")
    rows = (r0_v, r1_v, r2_v, r3_v)
    sgs = (sg0, sg1, sg2, sg3)
    sss = (ss0, ss1, ss2, ss3)

    zeros16 = jnp.zeros((L,), jnp.float32)
    iota16 = lax.iota(jnp.int32, L)
    cols = D // L

    base = c * (E // NC) + s * EPT
    pltpu.sync_copy(src_hbm.at[pl.ds(base, EPT)], lo_v.at[pl.ds(0, EPT)])
    pltpu.sync_copy(dst_hbm.at[pl.ds(base, EPT)], hi_v.at[pl.ds(0, EPT)])

    def scan_body(i, carry):
        cl, ch = carry
        sv = lo_v[pl.ds(i * L, L)]
        dv = hi_v[pl.ds(i * L, L)]
        packed = (sv << PBITS) | dv
        mlo = dv < HALF
        ones = jnp.where(mlo, 1, 0).astype(jnp.int32)
        pfx = plsc.cumsum(ones)
        tot = jnp.sum(ones)
        plsc.store_scatter(lo_v, [cl + pfx - 1], packed, mask=mlo)
        plsc.store_scatter(hi_v, [ch + iota16 - pfx], packed,
                           mask=jnp.logical_not(mlo))
        return (cl + tot, ch + (L - tot))

    cl, ch = lax.fori_loop(0, EPT // L, scan_body,
                           (jnp.int32(0), jnp.int32(0)))

    def pad_list(buf, cnt, trash_packed):
        def fix(k, carry):
            v = buf[pl.ds(k * L, L)]
            buf[pl.ds(k * L, L)] = jnp.where(k * L + iota16 < cnt,
                                             v, trash_packed)
            return carry
        lax.fori_loop(cnt // L, ((cnt + BB - 1) // BB) * (BB // L), fix, 0)

    pad_list(lo_v, cl, TRASH)
    pad_list(hi_v, ch, HALF + TRASH)

    for h, (listbuf, cnt, loadj) in enumerate(((lo_v, cl, 0),
                                               (hi_v, ch, HALF))):
        nb = (cnt + BB - 1) // BB

        def zero_r0(t, carry):
            r0_v[t // cols, pl.ds((t % cols) * L, L)] = zeros16
            return carry

        lax.fori_loop(0, BB * cols, zero_r0, 0)
        for k in range(AGG // NS // ZR):
            pltpu.sync_copy(r0_v.at[pl.ds(0, ZR)],
                            shared.at[pl.ds(s * (AGG // NS) + k * ZR, ZR)])
        plsc.subcore_barrier()

        def unpack_issue(b, k):
            for q in range(BB // L):
                pk = listbuf[pl.ds(b * BB + q * L, L)]
                srcB[k, pl.ds(q * L, L)] = pk >> PBITS
                dstB[k, pl.ds(q * L, L)] = (pk & PMASK) - loadj
            pltpu.async_copy(xw_hbm.at[srcB.at[k]], rows[k], sgs[k])

        for k in range(3):
            @pl.when(k < nb)
            def _(k=k):
                unpack_issue(k, k)

        def sup(j4, carry):
            for k in range(4):
                b = j4 * 4 + k
                kn = (k + 3) % 4

                @pl.when(b < nb)
                def _(b=b, k=k, kn=kn):
                    pltpu.make_async_copy(xw_hbm.at[srcB.at[k]],
                                          rows[k], sgs[k]).wait()
                    pltpu.async_copy(rows[k], shared.at[dstB.at[k]],
                                     sss[k], add=True)

                    @pl.when(b + 3 < nb)
                    def _(b=b, k=k, kn=kn):
                        @pl.when(b >= 1)
                        def _(kn=kn):
                            pltpu.make_async_copy(
                                rows[kn], shared.at[dstB.at[kn]],
                                sss[kn]).wait()
                        unpack_issue(b + 3, kn)
            return carry

        lax.fori_loop(0, (nb + 3) // 4, sup, 0)

        for k in range(4):
            @pl.when(k < nb)
            def _(k=k):
                pltpu.make_async_copy(rows[k], shared.at[dstB.at[k]],
                                      sss[k]).wait()
        plsc.subcore_barrier()

        for k in range(NS):
            @pl.when(s == k)
            def _(k=k, h=h):
                pltpu.sync_copy(shared.at[pl.ds(k * OPT, OPT)],
                                out_hbm.at[h, c, pl.ds(k * OPT, OPT), :])
        plsc.subcore_barrier()


def _ep_body(p_ref, deg_ref, b_ref, out_ref):
    scale = lax.rsqrt(jnp.maximum(deg_ref[...], 1.0))
    p = p_ref[...]
    agg = (p[0, 0] + p[0, 1]) * scale
    out_ref[...] = jnp.maximum(agg + b_ref[...], 0.0)


def _epilogue(p, deg_in2d, b2d):
    return pl.pallas_call(
        _ep_body,
        grid=(N // RMM,),
        in_specs=[
            pl.BlockSpec((1, NC, RMM, D), lambda i: (i // 5, 0, i % 5, 0)),
            pl.BlockSpec((RMM, 1), lambda i: (i, 0)),
            pl.BlockSpec((1, D), lambda i: (0, 0)),
        ],
        out_specs=pl.BlockSpec((RMM, D), lambda i: (i, 0)),
        out_shape=jax.ShapeDtypeStruct((N, D), jnp.float32),
    )(p, deg_in2d, b2d)


def kernel(features, edge_index, W, b):
    srcf = edge_index[0]
    dstf = edge_index[1]
    dsrc, ddst = _deg_kernel(srcf, dstf)
    deg_out2d = dsrc[:N, None]
    deg_in2d = ddst[:N, None]
    xw = _mm(features, W, deg_out2d)
    p = _gs_kernel(xw, srcf, dstf)
    return _epilogue(p, deg_in2d, b[None, :])


# speedup vs baseline: 1.0284x; 1.0284x over previous
import functools

import jax
import jax.numpy as jnp
from jax import lax
from jax.experimental import pallas as pl
from jax.experimental.pallas import tpu as pltpu
from jax.experimental.pallas import tpu_sc as plsc

N = 10000
E = 320000
D = 128

NC = 2
NS = 16
L = 16

_mesh = plsc.VectorSubcoreMesh(core_axis_name="c", subcore_axis_name="s")
_sc_params = pltpu.CompilerParams(needs_layout_passes=False)

HSZ = 10240
EPT_DEG = E // NS
HPT = HSZ // NS


@functools.partial(
    pl.kernel,
    out_type=[
        jax.ShapeDtypeStruct((HSZ,), jnp.float32),
        jax.ShapeDtypeStruct((HSZ,), jnp.float32),
    ],
    mesh=_mesh,
    scratch_types=[
        pltpu.VMEM((EPT_DEG,), jnp.int32),
        pltpu.VMEM((HSZ,), jnp.float32),
        pltpu.VMEM((HPT,), jnp.float32),
        pltpu.VMEM((HPT,), jnp.float32),
        pltpu.VMEM_SHARED((NS * HSZ,), jnp.float32),
    ],
    compiler_params=_sc_params,
)
def _deg_kernel(src_hbm, dst_hbm, dsrc_hbm, ddst_hbm,
                idx_v, hist_v, acc_v, tmp_v, shared):
    c = lax.axis_index("c")
    s = lax.axis_index("s")

    zeros16 = jnp.zeros((L,), jnp.float32)
    ones16 = jnp.ones((L,), jnp.float32)

    def zero_hist(i, carry):
        hist_v[pl.ds(i * L, L)] = zeros16
        return carry

    lax.fori_loop(0, HSZ // L, zero_hist, 0)

    @pl.when(c == 0)
    def _():
        pltpu.sync_copy(src_hbm.at[pl.ds(s * EPT_DEG, EPT_DEG)], idx_v)

    @pl.when(c == 1)
    def _():
        pltpu.sync_copy(dst_hbm.at[pl.ds(s * EPT_DEG, EPT_DEG)], idx_v)

    def accum(i, carry):
        base = i * (2 * L)
        for u in range(2):
            idx = idx_v[pl.ds(base + u * L, L)]
            plsc.addupdate_scatter(hist_v, [idx], ones16)
        return carry

    lax.fori_loop(0, EPT_DEG // (2 * L), accum, 0)

    pltpu.sync_copy(hist_v, shared.at[pl.ds(s * HSZ, HSZ)])
    plsc.subcore_barrier()

    def zero_acc(i, carry):
        acc_v[pl.ds(i * L, L)] = zeros16
        return carry

    lax.fori_loop(0, HPT // L, zero_acc, 0)

    def combine(k, carry):
        pltpu.sync_copy(shared.at[pl.ds(k * HSZ + s * HPT, HPT)], tmp_v)

        def add_vec(i, carry2):
            j = i * L
            acc_v[pl.ds(j, L)] = acc_v[pl.ds(j, L)] + tmp_v[pl.ds(j, L)]
            return carry2

        lax.fori_loop(0, HPT // L, add_vec, 0)
        return carry

    lax.fori_loop(0, NS, combine, 0)

    @pl.when(c == 0)
    def _():
        pltpu.sync_copy(acc_v, dsrc_hbm.at[pl.ds(s * HPT, HPT)])

    @pl.when(c == 1)
    def _():
        pltpu.sync_copy(acc_v, ddst_hbm.at[pl.ds(s * HPT, HPT)])


RMM = 1000


def _mm_body(f_ref, w_ref, deg_ref, xw_ref):
    scale = lax.rsqrt(jnp.maximum(deg_ref[...], 1.0))
    xw_ref[...] = jnp.dot(f_ref[...], w_ref[...],
                          preferred_element_type=jnp.float32) * scale


def _mm(features, W, deg_out2d):
    return pl.pallas_call(
        _mm_body,
        grid=(N // RMM,),
        in_specs=[
            pl.BlockSpec((RMM, D), lambda i: (i, 0)),
            pl.BlockSpec((D, D), lambda i: (0, 0)),
            pl.BlockSpec((RMM, 1), lambda i: (i, 0)),
        ],
        out_specs=pl.BlockSpec((RMM, D), lambda i: (i, 0)),
        out_shape=jax.ShapeDtypeStruct((N, D), jnp.float32),
    )(features, W, deg_out2d)


BB = 80
EPT = E // (NC * NS)
CAP = EPT + 240
HALF = 5000
AGG = 6144
TRASH = 5632
ZR = 48
OPT = 5120 // NS
PBITS = 14
PMASK = (1 << PBITS) - 1


@functools.partial(
    pl.kernel,
    out_type=jax.ShapeDtypeStruct((2, NC, 5120, D), jnp.float32),
    mesh=_mesh,
    scratch_types=[
        pltpu.VMEM((CAP,), jnp.int32),
        pltpu.VMEM((CAP,), jnp.int32),
        pltpu.VMEM((4, BB), jnp.int32),
        pltpu.VMEM((4, BB), jnp.int32),
        pltpu.VMEM((BB, D), jnp.float32),
        pltpu.VMEM((BB, D), jnp.float32),
        pltpu.VMEM((BB, D), jnp.float32),
        pltpu.VMEM((BB, D), jnp.float32),
        pltpu.VMEM_SHARED((AGG, D), jnp.float32),
        pltpu.SemaphoreType.DMA,
        pltpu.SemaphoreType.DMA,
        pltpu.SemaphoreType.DMA,
        pltpu.SemaphoreType.DMA,
    ],
    compiler_params=_sc_params,
)
def _gs_kernel(xw_hbm, src_hbm, dst_hbm, out_hbm,
               lo_v, hi_v, srcB, dstB, r0_v, r1_v, r2_v, r3_v,
               shared, sg0, sg1, sg2, sg3):
    c = lax.axis_index("c")
    s = lax.axis_index("s")
    rows = (r0_v, r1_v, r2_v, r3_v)
    sgs = (sg0, sg1, sg2, sg3)

    zeros16 = jnp.zeros((L,), jnp.float32)
    iota16 = lax.iota(jnp.int32, L)
    cols = D // L

    base = c * (E // NC) + s * EPT
    pltpu.sync_copy(src_hbm.at[pl.ds(base, EPT)], lo_v.at[pl.ds(0, EPT)])
    pltpu.sync_copy(dst_hbm.at[pl.ds(base, EPT)], hi_v.at[pl.ds(0, EPT)])

    def scan_body(i, carry):
        cl, ch = carry
        sv = lo_v[pl.ds(i * L, L)]
        dv = hi_v[pl.ds(i * L, L)]
        packed = (sv << PBITS) | dv
        mlo = dv < HALF
        ones = jnp.where(mlo, 1, 0).astype(jnp.int32)
        pfx = plsc.cumsum(ones)
        tot = jnp.sum(ones)
        plsc.store_scatter(lo_v, [cl + pfx - 1], packed, mask=mlo)
        plsc.store_scatter(hi_v, [ch + iota16 - pfx], packed,
                           mask=jnp.logical_not(mlo))
        return (cl + tot, ch + (L - tot))

    cl, ch = lax.fori_loop(0, EPT // L, scan_body,
                           (jnp.int32(0), jnp.int32(0)))

    def pad_list(buf, cnt, trash_packed):
        def fix(k, carry):
            v = buf[pl.ds(k * L, L)]
            buf[pl.ds(k * L, L)] = jnp.where(k * L + iota16 < cnt,
                                             v, trash_packed)
            return carry
        lax.fori_loop(cnt // L, ((cnt + BB - 1) // BB) * (BB // L), fix, 0)

    pad_list(lo_v, cl, TRASH)
    pad_list(hi_v, ch, HALF + TRASH)

    for h, (listbuf, cnt, loadj) in enumerate(((lo_v, cl, 0),
                                               (hi_v, ch, HALF))):
        nb = (cnt + BB - 1) // BB

        def zero_r0(t, carry):
            r0_v[t // cols, pl.ds((t % cols) * L, L)] = zeros16
            return carry

        lax.fori_loop(0, BB * cols, zero_r0, 0)
        for k in range(AGG // NS // ZR):
            pltpu.sync_copy(r0_v.at[pl.ds(0, ZR)],
                            shared.at[pl.ds(s * (AGG // NS) + k * ZR, ZR)])
        plsc.subcore_barrier()

        def unpack_issue(b, k):
            for q in range(BB // L):
                pk = listbuf[pl.ds(b * BB + q * L, L)]
                srcB[k, pl.ds(q * L, L)] = pk >> PBITS
                dstB[k, pl.ds(q * L, L)] = (pk & PMASK) - loadj
            pltpu.async_copy(xw_hbm.at[srcB.at[k]], rows[k], sgs[k])

        for k in range(4):
            @pl.when(k < nb)
            def _(k=k):
                unpack_issue(k, k)

        def sup(j4, carry):
            for k in range(4):
                b = j4 * 4 + k

                @pl.when(b < nb)
                def _(b=b, k=k):
                    pltpu.make_async_copy(xw_hbm.at[srcB.at[k]],
                                          rows[k], sgs[k]).wait()
                    pltpu.sync_copy(rows[k], shared.at[dstB.at[k]],
                                    add=True)

                    @pl.when(b + 4 < nb)
                    def _(b=b, k=k):
                        unpack_issue(b + 4, k)
            return carry

        lax.fori_loop(0, (nb + 3) // 4, sup, 0)
        plsc.subcore_barrier()

        for k in range(NS):
            @pl.when(s == k)
            def _(k=k, h=h):
                pltpu.sync_copy(shared.at[pl.ds(k * OPT, OPT)],
                                out_hbm.at[h, c, pl.ds(k * OPT, OPT), :])
        plsc.subcore_barrier()


def _ep_body(p_ref, deg_ref, b_ref, out_ref):
    scale = lax.rsqrt(jnp.maximum(deg_ref[...], 1.0))
    p = p_ref[...]
    agg = (p[0, 0] + p[0, 1]) * scale
    out_ref[...] = jnp.maximum(agg + b_ref[...], 0.0)


def _epilogue(p, deg_in2d, b2d):
    return pl.pallas_call(
        _ep_body,
        grid=(N // RMM,),
        in_specs=[
            pl.BlockSpec((1, NC, RMM, D), lambda i: (i // 5, 0, i % 5, 0)),
            pl.BlockSpec((RMM, 1), lambda i: (i, 0)),
            pl.BlockSpec((1, D), lambda i: (0, 0)),
        ],
        out_specs=pl.BlockSpec((RMM, D), lambda i: (i, 0)),
        out_shape=jax.ShapeDtypeStruct((N, D), jnp.float32),
    )(p, deg_in2d, b2d)


def kernel(features, edge_index, W, b):
    srcf = edge_index[0]
    dstf = edge_index[1]
    dsrc, ddst = _deg_kernel(srcf, dstf)
    deg_out2d = dsrc[:N, None]
    deg_in2d = ddst[:N, None]
    xw = _mm(features, W, deg_out2d)
    p = _gs_kernel(xw, srcf, dstf)
    return _epilogue(p, deg_in2d, b[None, :])
